# baseline (speedup 1.0000x reference)
import functools

import jax
import jax.numpy as jnp
from jax import lax
from jax.experimental import pallas as pl
from jax.experimental.pallas import tpu as pltpu
from jax.experimental.pallas import tpu_sc as plsc

N = 10000
E = 320000
D = 128
DE = 16
H = 8
C = 16
L = 6
NP = 10240

NC = 2
NS = 16
NW = NC * NS
CPT = D // NW
CH = 2000
NCHUNK = E // CH
GRP = CH // 16
ECH = 6400
ENCHUNK = E // ECH
EGRP = ECH // 16

_mesh = plsc.VectorSubcoreMesh(
    core_axis_name="c", subcore_axis_name="s", num_cores=NC, num_subcores=NS)


@functools.partial(
    pl.kernel,
    out_type=[jax.ShapeDtypeStruct((D * NP,), jnp.float32),
              jax.ShapeDtypeStruct((NW * NP,), jnp.float32)],
    mesh=_mesh,
    compiler_params=pltpu.CompilerParams(needs_layout_passes=False),
    scratch_types=[
        pltpu.VMEM(((CPT // 2) * NP,), jnp.int32),
        pltpu.VMEM((CPT * NP,), jnp.float32),
        pltpu.VMEM((NP,), jnp.float32),
        pltpu.VMEM((NP,), jnp.float32),
        pltpu.VMEM((NP,), jnp.float32),
        pltpu.VMEM((ECH,), jnp.int32),
        pltpu.VMEM((ECH,), jnp.float32),
        pltpu.VMEM((ECH,), jnp.int32),
        pltpu.VMEM((ECH,), jnp.float32),
        pltpu.VMEM((ECH,), jnp.float32),
        pltpu.SemaphoreType.DMA,
        pltpu.SemaphoreType.DMA,
        pltpu.SemaphoreType.DMA,
    ],
)
def _edge_sc(hP, asT, adT, aeT, sdE, acc_out, den_out,
             h_l, acc_l, as_l, ad_l, den_l,
             sd0, a0, sd1, a1, t_buf, sem0, sem1, semi):
    w = lax.axis_index("s") * NC + lax.axis_index("c")
    hd = w // (NW // H)
    blk = w % (NW // H)

    pltpu.async_copy(hP.at[pl.ds(w * ((CPT // 2) * NP), (CPT // 2) * NP)], h_l, semi)
    pltpu.async_copy(asT.at[pl.ds(hd * NP, NP)], as_l, semi)
    pltpu.async_copy(adT.at[pl.ds(hd * NP, NP)], ad_l, semi)

    zf = jnp.zeros((16,), jnp.float32)

    def zero_acc(i, _):
        acc_l[pl.ds(i * 16, 16)] = zf
        return 0
    lax.fori_loop(0, CPT * NP // 16, zero_acc, 0)

    def zero_den(i, _):
        den_l[pl.ds(i * 16, 16)] = zf
        return 0
    lax.fori_loop(0, NP // 16, zero_den, 0)

    pltpu.make_async_copy(hP.at[pl.ds(0, (CPT // 2) * NP)], h_l, semi).wait()
    pltpu.make_async_copy(asT.at[pl.ds(0, NP)], as_l, semi).wait()
    pltpu.make_async_copy(adT.at[pl.ds(0, NP)], ad_l, semi).wait()

    def start_chunk(c, sd, a, sem):
        pltpu.async_copy(sdE.at[pl.ds(c * ECH, ECH)], sd, sem)
        pltpu.async_copy(aeT.at[pl.ds(hd * E + c * ECH, ECH)], a, sem)

    def wait_chunk(sd, a, sem):
        pltpu.make_async_copy(sdE.at[pl.ds(0, ECH)], sd, sem).wait()
        pltpu.make_async_copy(aeT.at[pl.ds(0, ECH)], a, sem).wait()

    def process(sd, a, mine):
        @plsc.parallel_loop(0, EGRP, 1, unroll=2)
        def tbody(g):
            sd16 = sd[pl.ds(g * 16, 16)]
            s16 = sd16 & 0xFFFF
            d16 = lax.shift_right_logical(sd16, 16)
            ae16 = a[pl.ds(g * 16, 16)]
            z = plsc.load_gather(as_l, [s16]) + plsc.load_gather(ad_l, [d16]) + ae16
            t = jnp.exp(jnp.maximum(z, 0.2 * z))
            t_buf[pl.ds(g * 16, 16)] = t

            @pl.when(mine)
            def _():
                plsc.addupdate_scatter(den_l, [d16], t)

        @plsc.parallel_loop(0, EGRP, 1, unroll=2)
        def mbody(g):
            sd16 = sd[pl.ds(g * 16, 16)]
            s16 = sd16 & 0xFFFF
            d16 = lax.shift_right_logical(sd16, 16)
            t = t_buf[pl.ds(g * 16, 16)]
            for r in range(CPT // 2):
                wv = plsc.load_gather(h_l, [s16 + (r * NP)])
                hb = plsc.bitcast(wv, jnp.bfloat16)
                hlo, hhi = plsc.unpack(hb, format=plsc.PackFormat.INTERLEAVED)
                plsc.addupdate_scatter(acc_l, [d16 + ((2 * r) * NP)], t * hlo)
                plsc.addupdate_scatter(acc_l, [d16 + ((2 * r + 1) * NP)], t * hhi)

    start_chunk(0, sd0, a0, sem0)

    def chunk_body(c, _):
        @pl.when(c % 2 == 0)
        def _():
            wait_chunk(sd0, a0, sem0)

            @pl.when(c + 1 < ENCHUNK)
            def _():
                start_chunk(c + 1, sd1, a1, sem1)
            process(sd0, a0, c % (NW // H) == blk)

        @pl.when(c % 2 == 1)
        def _():
            wait_chunk(sd1, a1, sem1)

            @pl.when(c + 1 < ENCHUNK)
            def _():
                start_chunk(c + 1, sd0, a0, sem0)
            process(sd1, a1, c % (NW // H) == blk)
        return 0
    lax.fori_loop(0, ENCHUNK, chunk_body, 0)

    pltpu.sync_copy(acc_l, acc_out.at[pl.ds(w * (CPT * NP), CPT * NP)])
    pltpu.sync_copy(den_l, den_out.at[pl.ds(w * NP, NP)])


@functools.partial(
    pl.kernel,
    out_type=jax.ShapeDtypeStruct(((DE + 1) * NP,), jnp.float32),
    mesh=_mesh,
    compiler_params=pltpu.CompilerParams(needs_layout_passes=False),
    scratch_types=[
        pltpu.VMEM((NP,), jnp.float32),
        pltpu.VMEM((CH,), jnp.int32),
        pltpu.VMEM((CH,), jnp.float32),
        pltpu.VMEM((CH,), jnp.int32),
        pltpu.VMEM((CH,), jnp.float32),
        pltpu.SemaphoreType.DMA,
        pltpu.SemaphoreType.DMA,
    ],
)
def _loopstat_sc(eaT17, dstE, out, acc, d0, v0, d1, v1, sem0, sem1):
    w = lax.axis_index("s") * NC + lax.axis_index("c")

    @pl.when(w < DE + 1)
    def _():
        zf = jnp.zeros((16,), jnp.float32)

        def zero_body(i, _):
            acc[pl.ds(i * 16, 16)] = zf
            return 0
        lax.fori_loop(0, NP // 16, zero_body, 0)

        def start_chunk(c, d, v, sem):
            pltpu.async_copy(dstE.at[pl.ds(c * CH, CH)], d, sem)
            pltpu.async_copy(eaT17.at[pl.ds(w * E + c * CH, CH)], v, sem)

        def wait_chunk(d, v, sem):
            pltpu.make_async_copy(dstE.at[pl.ds(0, CH)], d, sem).wait()
            pltpu.make_async_copy(eaT17.at[pl.ds(0, CH)], v, sem).wait()

        def process(d, v):
            @plsc.parallel_loop(0, GRP, 1, unroll=4)
            def body(g):
                d16 = d[pl.ds(g * 16, 16)]
                v16 = v[pl.ds(g * 16, 16)]
                plsc.addupdate_scatter(acc, [d16], v16)

        start_chunk(0, d0, v0, sem0)

        def chunk_body(c, _):
            @pl.when(c % 2 == 0)
            def _():
                wait_chunk(d0, v0, sem0)

                @pl.when(c + 1 < NCHUNK)
                def _():
                    start_chunk(c + 1, d1, v1, sem1)
                process(d0, v0)

            @pl.when(c % 2 == 1)
            def _():
                wait_chunk(d1, v1, sem1)

                @pl.when(c + 1 < NCHUNK)
                def _():
                    start_chunk(c + 1, d0, v0, sem0)
                process(d1, v1)
            return 0
        lax.fori_loop(0, NCHUNK, chunk_body, 0)

        pltpu.sync_copy(acc, out.at[pl.ds(w * NP, NP)])


_EB = 3200


def _ae_prep_body(pt_ref, ea_ref, out_ref):
    out_ref[...] = jnp.dot(pt_ref[...], ea_ref[...],
                           preferred_element_type=jnp.float32)


def _ae_prep(Pt, eaT):
    return pl.pallas_call(
        _ae_prep_body,
        grid=(E // _EB,),
        in_specs=[pl.BlockSpec((H, DE), lambda i: (0, 0)),
                  pl.BlockSpec((DE, _EB), lambda i: (0, i))],
        out_specs=pl.BlockSpec((H, _EB), lambda i: (0, i)),
        out_shape=jax.ShapeDtypeStruct((H, E), jnp.float32),
    )(Pt, eaT)


def _loopprep_body(pt_ref, st_ref, out_ref):
    sums = st_ref[:DE, :]
    cnt = st_ref[DE:, :]
    la = sums / jnp.maximum(cnt, 1.0)
    out_ref[...] = jnp.dot(pt_ref[...], la, preferred_element_type=jnp.float32)


def _loopprep(Pt, stats):
    return pl.pallas_call(
        _loopprep_body,
        in_specs=[pl.BlockSpec((H, DE), lambda: (0, 0)),
                  pl.BlockSpec((DE + 1, NP), lambda: (0, 0))],
        out_specs=pl.BlockSpec((H, NP), lambda: (0, 0)),
        out_shape=jax.ShapeDtypeStruct((H, NP), jnp.float32),
    )(Pt, stats)


_NB = 2048


def _pack_h(h):
    hb = h.astype(jnp.bfloat16).reshape(D // 2, 2, h.shape[-1])
    lo = lax.bitcast_convert_type(hb[:, 0, :], jnp.uint16).astype(jnp.uint32)
    hi = lax.bitcast_convert_type(hb[:, 1, :], jnp.uint16).astype(jnp.uint32)
    return lax.bitcast_convert_type(lo | (hi << 16), jnp.int32)


def _head_body(wt_ref, as_ref, ad_ref, xt_ref, h_ref, hp_ref, a_ref, b_ref):
    h = jnp.dot(wt_ref[...], xt_ref[...], preferred_element_type=jnp.float32)
    h_ref[...] = h
    hp_ref[...] = _pack_h(h)
    a_ref[...] = jnp.dot(as_ref[...], h, preferred_element_type=jnp.float32)
    b_ref[...] = jnp.dot(ad_ref[...], h, preferred_element_type=jnp.float32)


def _head(WT, As, Ad, xT):
    nspec = pl.BlockSpec((D, _NB), lambda i: (0, i))
    pspec = pl.BlockSpec((D // 2, _NB), lambda i: (0, i))
    hspec = pl.BlockSpec((H, _NB), lambda i: (0, i))
    return pl.pallas_call(
        _head_body,
        grid=(NP // _NB,),
        in_specs=[pl.BlockSpec((D, D), lambda i: (0, 0)),
                  pl.BlockSpec((H, D), lambda i: (0, 0)),
                  pl.BlockSpec((H, D), lambda i: (0, 0)),
                  nspec],
        out_specs=[nspec, pspec, hspec, hspec],
        out_shape=[jax.ShapeDtypeStruct((D, NP), jnp.float32),
                   jax.ShapeDtypeStruct((D // 2, NP), jnp.int32),
                   jax.ShapeDtypeStruct((H, NP), jnp.float32),
                   jax.ShapeDtypeStruct((H, NP), jnp.float32)],
    )(WT, As, Ad, xT)


def _ln0(x, g, b):
    mu = jnp.mean(x, axis=0, keepdims=True)
    var = jnp.mean((x - mu) ** 2, axis=0, keepdims=True)
    return (x - mu) * lax.rsqrt(var + 1e-5) * g + b


def _combine_body(mT_ref, hT_ref, acc_ref, den_ref, r832_ref, as_ref, ad_ref, ael_ref,
                  rmat_ref, bias_ref, w1t_ref, b1_ref, w2t_ref, b2_ref,
                  ln1g_ref, ln1b_ref, ln2g_ref, ln2b_ref,
                  wt_ref, asm_ref, adm_ref,
                  out_ref, hn_ref, hp_ref, an_ref, bn_ref):
    z = as_ref[...] + ad_ref[...] + ael_ref[...]
    tl = jnp.exp(jnp.maximum(z, 0.2 * z))
    den8 = jnp.dot(r832_ref[...], den_ref[...],
                   preferred_element_type=jnp.float32) + tl
    rmat = rmat_ref[...]
    dfull = jnp.dot(rmat, den8, preferred_element_type=jnp.float32)
    tfull = jnp.dot(rmat, tl, preferred_element_type=jnp.float32)
    g = (acc_ref[...] + tfull * hT_ref[...]) / (dfull + 1e-16) + bias_ref[...]
    t = _ln0(g + mT_ref[...], ln1g_ref[...], ln1b_ref[...])
    f = jnp.maximum(jnp.dot(w1t_ref[...], t, preferred_element_type=jnp.float32)
                    + b1_ref[...], 0.0)
    y = jnp.dot(w2t_ref[...], f, preferred_element_type=jnp.float32) + b2_ref[...]
    out = _ln0(y + t, ln2g_ref[...], ln2b_ref[...])
    out_ref[...] = out
    hn = jnp.dot(wt_ref[...], out, preferred_element_type=jnp.float32)
    hn_ref[...] = hn
    hp_ref[...] = _pack_h(hn)
    an_ref[...] = jnp.dot(asm_ref[...], hn, preferred_element_type=jnp.float32)
    bn_ref[...] = jnp.dot(adm_ref[...], hn, preferred_element_type=jnp.float32)


def _combine(mT, hT, accT, denT, R832, asT, adT, aelT, Rmat, bias,
             w1T, b1, w2T, b2, ln1g, ln1b, ln2g, ln2b, WT, As, Ad):
    nspec = pl.BlockSpec((D, _NB), lambda i: (0, i))
    hspec = pl.BlockSpec((H, _NB), lambda i: (0, i))
    cvec = pl.BlockSpec((D, 1), lambda i: (0, 0))
    return pl.pallas_call(
        _combine_body,
        grid=(NP // _NB,),
        in_specs=[nspec, nspec, nspec,
                  pl.BlockSpec((NW, _NB), lambda i: (0, i)),
                  pl.BlockSpec((H, NW), lambda i: (0, 0)),
                  hspec, hspec, hspec,
                  pl.BlockSpec((D, H), lambda i: (0, 0)),
                  cvec,
                  pl.BlockSpec((4 * D, D), lambda i: (0, 0)),
                  pl.BlockSpec((4 * D, 1), lambda i: (0, 0)),
                  pl.BlockSpec((D, 4 * D), lambda i: (0, 0)),
                  cvec, cvec, cvec, cvec, cvec,
                  pl.BlockSpec((D, D), lambda i: (0, 0)),
                  pl.BlockSpec((H, D), lambda i: (0, 0)),
                  pl.BlockSpec((H, D), lambda i: (0, 0))],
        out_specs=[nspec, nspec, pl.BlockSpec((D // 2, _NB), lambda i: (0, i)),
                   hspec, hspec],
        out_shape=[jax.ShapeDtypeStruct((D, NP), jnp.float32),
                   jax.ShapeDtypeStruct((D, NP), jnp.float32),
                   jax.ShapeDtypeStruct((D // 2, NP), jnp.int32),
                   jax.ShapeDtypeStruct((H, NP), jnp.float32),
                   jax.ShapeDtypeStruct((H, NP), jnp.float32)],
    )(mT, hT, accT, denT, R832, asT, adT, aelT, Rmat, bias, w1T, b1, w2T, b2,
      ln1g, ln1b, ln2g, ln2b, WT, As, Ad)


def kernel(x, edge_index, edge_attr, gat_W, gat_att_src, gat_att_dst,
           gat_lin_edge, gat_att_edge, gat_bias, ffn_w1, ffn_b1, ffn_w2,
           ffn_b2, ln1_g, ln1_b, ln2_g, ln2_b):
    src = edge_index[0]
    dst = edge_index[1]
    sdE = jnp.bitwise_or(src, jnp.left_shift(dst, 16))
    xT = jnp.zeros((D, NP), jnp.float32).at[:, :N].set(x.T)
    eaT = edge_attr.T
    eaT17 = jnp.concatenate([eaT, jnp.ones((1, E), jnp.float32)], axis=0).reshape((DE + 1) * E)

    WT = gat_W.T
    le_r = gat_lin_edge.reshape(DE, H, C)
    Pt = jnp.einsum('dhc,hc->hd', le_r, gat_att_edge[0])
    eye8 = jnp.eye(H, dtype=jnp.float32)
    As = (eye8[:, :, None] * gat_att_src[0][:, None, :]).reshape(H, D)
    Ad = (eye8[:, :, None] * gat_att_dst[0][:, None, :]).reshape(H, D)
    Rmat = jnp.repeat(eye8, C, axis=0)
    R832 = jnp.repeat(eye8, NW // H, axis=1)
    bias = gat_bias.reshape(D, 1)

    aeT = _ae_prep(Pt, eaT).reshape(H * E)
    stats = _loopstat_sc(eaT17, dst).reshape(DE + 1, NP)
    aelT = _loopprep(Pt, stats)

    mT = xT
    hT, hP, asT, adT = _head(WT, As, Ad, xT)
    for i in range(L):
        accf, denf = _edge_sc(hP.reshape((D // 2) * NP), asT.reshape(H * NP),
                              adT.reshape(H * NP), aeT, sdE)
        accT = accf.reshape(D, NP)
        denT = denf.reshape(NW, NP)
        mT, hT, hP, asT, adT = _combine(
            mT, hT, accT, denT, R832, asT, adT, aelT, Rmat, bias,
            ffn_w1[i].T, ffn_b1[i].reshape(4 * D, 1),
            ffn_w2[i].T, ffn_b2[i].reshape(D, 1),
            ln1_g[i].reshape(D, 1), ln1_b[i].reshape(D, 1),
            ln2_g[i].reshape(D, 1), ln2_b[i].reshape(D, 1),
            WT, As, Ad)
    return mT[:, :N].T


# speedup vs baseline: 1.1855x; 1.1855x over previous
import functools

import jax
import jax.numpy as jnp
from jax import lax
from jax.experimental import pallas as pl
from jax.experimental.pallas import tpu as pltpu
from jax.experimental.pallas import tpu_sc as plsc

N = 10000
E = 320000
D = 128
DE = 16
H = 8
C = 16
L = 6
NP = 10240

NC = 2
NS = 16
NW = NC * NS
CPT = D // NW
CH = 2000
NCHUNK = E // CH
GRP = CH // 16
ECH = 8000
ENCHUNK = E // ECH
EGRP = ECH // 16

_mesh = plsc.VectorSubcoreMesh(
    core_axis_name="c", subcore_axis_name="s", num_cores=NC, num_subcores=NS)


@functools.partial(
    pl.kernel,
    out_type=[jax.ShapeDtypeStruct((D * NP,), jnp.float32),
              jax.ShapeDtypeStruct((NW * NP,), jnp.float32)],
    mesh=_mesh,
    compiler_params=pltpu.CompilerParams(needs_layout_passes=False),
    scratch_types=[
        pltpu.VMEM(((CPT // 2) * NP,), jnp.int32),
        pltpu.VMEM((CPT * NP,), jnp.float32),
        pltpu.VMEM((NP,), jnp.float32),
        pltpu.VMEM((NP,), jnp.float32),
        pltpu.VMEM((NP,), jnp.float32),
        pltpu.VMEM((ECH,), jnp.int32),
        pltpu.VMEM((ECH,), jnp.float32),
        pltpu.VMEM((ECH,), jnp.int32),
        pltpu.VMEM((ECH,), jnp.float32),
        pltpu.SemaphoreType.DMA,
        pltpu.SemaphoreType.DMA,
        pltpu.SemaphoreType.DMA,
    ],
)
def _edge_sc(hP, asT, adT, aeT, sdE, acc_out, den_out,
             h_l, acc_l, as_l, ad_l, den_l,
             sd0, a0, sd1, a1, sem0, sem1, semi):
    w = lax.axis_index("s") * NC + lax.axis_index("c")
    hd = w // (NW // H)
    blk = w % (NW // H)

    pltpu.async_copy(hP.at[pl.ds(w * ((CPT // 2) * NP), (CPT // 2) * NP)], h_l, semi)
    pltpu.async_copy(asT.at[pl.ds(hd * NP, NP)], as_l, semi)
    pltpu.async_copy(adT.at[pl.ds(hd * NP, NP)], ad_l, semi)

    zf = jnp.zeros((16,), jnp.float32)

    def zero_acc(i, _):
        acc_l[pl.ds(i * 16, 16)] = zf
        return 0
    lax.fori_loop(0, CPT * NP // 16, zero_acc, 0)

    def zero_den(i, _):
        den_l[pl.ds(i * 16, 16)] = zf
        return 0
    lax.fori_loop(0, NP // 16, zero_den, 0)

    pltpu.make_async_copy(hP.at[pl.ds(0, (CPT // 2) * NP)], h_l, semi).wait()
    pltpu.make_async_copy(asT.at[pl.ds(0, NP)], as_l, semi).wait()
    pltpu.make_async_copy(adT.at[pl.ds(0, NP)], ad_l, semi).wait()

    def start_chunk(c, sd, a, sem):
        pltpu.async_copy(sdE.at[pl.ds(c * ECH, ECH)], sd, sem)
        pltpu.async_copy(aeT.at[pl.ds(hd * E + c * ECH, ECH)], a, sem)

    def wait_chunk(sd, a, sem):
        pltpu.make_async_copy(sdE.at[pl.ds(0, ECH)], sd, sem).wait()
        pltpu.make_async_copy(aeT.at[pl.ds(0, ECH)], a, sem).wait()

    def process(sd, a, mine):
        @plsc.parallel_loop(0, EGRP, 1, unroll=2)
        def body(g):
            sd16 = sd[pl.ds(g * 16, 16)]
            s16 = sd16 & 0xFFFF
            d16 = lax.shift_right_logical(sd16, 16)
            ae16 = a[pl.ds(g * 16, 16)]
            z = plsc.load_gather(as_l, [s16]) + plsc.load_gather(ad_l, [d16]) + ae16
            t = jnp.exp(jnp.maximum(z, 0.2 * z))

            @pl.when(mine)
            def _():
                plsc.addupdate_scatter(den_l, [d16], t)

            for r in range(CPT // 2):
                wv = plsc.load_gather(h_l, [s16 + (r * NP)])
                hb = plsc.bitcast(wv, jnp.bfloat16)
                hlo, hhi = plsc.unpack(hb, format=plsc.PackFormat.INTERLEAVED)
                plsc.addupdate_scatter(acc_l, [d16 + ((2 * r) * NP)], t * hlo)
                plsc.addupdate_scatter(acc_l, [d16 + ((2 * r + 1) * NP)], t * hhi)

    start_chunk(0, sd0, a0, sem0)

    def chunk_body(c, _):
        @pl.when(c % 2 == 0)
        def _():
            wait_chunk(sd0, a0, sem0)

            @pl.when(c + 1 < ENCHUNK)
            def _():
                start_chunk(c + 1, sd1, a1, sem1)
            process(sd0, a0, c % (NW // H) == blk)

        @pl.when(c % 2 == 1)
        def _():
            wait_chunk(sd1, a1, sem1)

            @pl.when(c + 1 < ENCHUNK)
            def _():
                start_chunk(c + 1, sd0, a0, sem0)
            process(sd1, a1, c % (NW // H) == blk)
        return 0
    lax.fori_loop(0, ENCHUNK, chunk_body, 0)

    pltpu.sync_copy(acc_l, acc_out.at[pl.ds(w * (CPT * NP), CPT * NP)])
    pltpu.sync_copy(den_l, den_out.at[pl.ds(w * NP, NP)])


LCH = 4000
LNCH = (E // 2) // LCH
LGRP = LCH // 16


@functools.partial(
    pl.kernel,
    out_type=jax.ShapeDtypeStruct(((NW + 2) * NP,), jnp.float32),
    mesh=_mesh,
    compiler_params=pltpu.CompilerParams(needs_layout_passes=False),
    scratch_types=[
        pltpu.VMEM((NP,), jnp.float32),
        pltpu.VMEM((NP,), jnp.float32),
        pltpu.VMEM((LCH,), jnp.int32),
        pltpu.VMEM((LCH,), jnp.float32),
        pltpu.VMEM((LCH,), jnp.int32),
        pltpu.VMEM((LCH,), jnp.float32),
        pltpu.SemaphoreType.DMA,
        pltpu.SemaphoreType.DMA,
    ],
)
def _loopstat_sc(eaTf, sdE, out, acc, cnt, sd0, v0, sd1, v1, sem0, sem1):
    w = lax.axis_index("s") * NC + lax.axis_index("c")
    ch = w % DE
    half = w // DE
    base = half * (E // 2)
    is_cnt = ch == 0

    zf = jnp.zeros((16,), jnp.float32)

    def zero_body(i, _):
        acc[pl.ds(i * 16, 16)] = zf
        cnt[pl.ds(i * 16, 16)] = zf
        return 0
    lax.fori_loop(0, NP // 16, zero_body, 0)

    def start_chunk(c, sd, v, sem):
        pltpu.async_copy(sdE.at[pl.ds(base + c * LCH, LCH)], sd, sem)
        pltpu.async_copy(eaTf.at[pl.ds(ch * E + base + c * LCH, LCH)], v, sem)

    def wait_chunk(sd, v, sem):
        pltpu.make_async_copy(sdE.at[pl.ds(0, LCH)], sd, sem).wait()
        pltpu.make_async_copy(eaTf.at[pl.ds(0, LCH)], v, sem).wait()

    ones16 = jnp.ones((16,), jnp.float32)

    def process(sd, v):
        @plsc.parallel_loop(0, LGRP, 1, unroll=2)
        def body(g):
            d16 = lax.shift_right_logical(sd[pl.ds(g * 16, 16)], 16)
            v16 = v[pl.ds(g * 16, 16)]
            plsc.addupdate_scatter(acc, [d16], v16)

            @pl.when(is_cnt)
            def _():
                plsc.addupdate_scatter(cnt, [d16], ones16)

    start_chunk(0, sd0, v0, sem0)

    def chunk_body(c, _):
        @pl.when(c % 2 == 0)
        def _():
            wait_chunk(sd0, v0, sem0)

            @pl.when(c + 1 < LNCH)
            def _():
                start_chunk(c + 1, sd1, v1, sem1)
            process(sd0, v0)

        @pl.when(c % 2 == 1)
        def _():
            wait_chunk(sd1, v1, sem1)

            @pl.when(c + 1 < LNCH)
            def _():
                start_chunk(c + 1, sd0, v0, sem0)
            process(sd1, v1)
        return 0
    lax.fori_loop(0, LNCH, chunk_body, 0)

    pltpu.sync_copy(acc, out.at[pl.ds(w * NP, NP)])

    @pl.when(is_cnt)
    def _():
        pltpu.sync_copy(cnt, out.at[pl.ds((NW + half) * NP, NP)])


_EB = 3200


def _ae_prep_body(pt_ref, ea_ref, out_ref):
    out_ref[...] = jnp.dot(pt_ref[...], ea_ref[...],
                           preferred_element_type=jnp.float32)


def _ae_prep(Pt, eaT):
    return pl.pallas_call(
        _ae_prep_body,
        grid=(E // _EB,),
        in_specs=[pl.BlockSpec((H, DE), lambda i: (0, 0)),
                  pl.BlockSpec((DE, _EB), lambda i: (0, i))],
        out_specs=pl.BlockSpec((H, _EB), lambda i: (0, i)),
        out_shape=jax.ShapeDtypeStruct((H, E), jnp.float32),
    )(Pt, eaT)


def _loopprep_body(pt_ref, st_ref, out_ref):
    sums = st_ref[:DE, :] + st_ref[DE:NW, :]
    cnt = st_ref[NW:NW + 1, :] + st_ref[NW + 1:, :]
    la = sums / jnp.maximum(cnt, 1.0)
    out_ref[...] = jnp.dot(pt_ref[...], la, preferred_element_type=jnp.float32)


def _loopprep(Pt, stats):
    return pl.pallas_call(
        _loopprep_body,
        in_specs=[pl.BlockSpec((H, DE), lambda: (0, 0)),
                  pl.BlockSpec((NW + 2, NP), lambda: (0, 0))],
        out_specs=pl.BlockSpec((H, NP), lambda: (0, 0)),
        out_shape=jax.ShapeDtypeStruct((H, NP), jnp.float32),
    )(Pt, stats)


_NB = 2048


def _pack_h(h):
    hb = h.astype(jnp.bfloat16).reshape(D // 2, 2, h.shape[-1])
    lo = lax.bitcast_convert_type(hb[:, 0, :], jnp.uint16).astype(jnp.uint32)
    hi = lax.bitcast_convert_type(hb[:, 1, :], jnp.uint16).astype(jnp.uint32)
    return lax.bitcast_convert_type(lo | (hi << 16), jnp.int32)


def _head_body(wt_ref, as_ref, ad_ref, xt_ref, h_ref, hp_ref, a_ref, b_ref):
    h = jnp.dot(wt_ref[...], xt_ref[...], preferred_element_type=jnp.float32)
    h_ref[...] = h
    hp_ref[...] = _pack_h(h)
    a_ref[...] = jnp.dot(as_ref[...], h, preferred_element_type=jnp.float32)
    b_ref[...] = jnp.dot(ad_ref[...], h, preferred_element_type=jnp.float32)


def _head(WT, As, Ad, xT):
    nspec = pl.BlockSpec((D, _NB), lambda i: (0, i))
    pspec = pl.BlockSpec((D // 2, _NB), lambda i: (0, i))
    hspec = pl.BlockSpec((H, _NB), lambda i: (0, i))
    return pl.pallas_call(
        _head_body,
        grid=(NP // _NB,),
        in_specs=[pl.BlockSpec((D, D), lambda i: (0, 0)),
                  pl.BlockSpec((H, D), lambda i: (0, 0)),
                  pl.BlockSpec((H, D), lambda i: (0, 0)),
                  nspec],
        out_specs=[nspec, pspec, hspec, hspec],
        out_shape=[jax.ShapeDtypeStruct((D, NP), jnp.float32),
                   jax.ShapeDtypeStruct((D // 2, NP), jnp.int32),
                   jax.ShapeDtypeStruct((H, NP), jnp.float32),
                   jax.ShapeDtypeStruct((H, NP), jnp.float32)],
    )(WT, As, Ad, xT)


def _ln0(x, g, b):
    mu = jnp.mean(x, axis=0, keepdims=True)
    var = jnp.mean((x - mu) ** 2, axis=0, keepdims=True)
    return (x - mu) * lax.rsqrt(var + 1e-5) * g + b


def _combine_body(mT_ref, hT_ref, acc_ref, den_ref, r832_ref, as_ref, ad_ref, ael_ref,
                  rmat_ref, bias_ref, w1t_ref, b1_ref, w2t_ref, b2_ref,
                  ln1g_ref, ln1b_ref, ln2g_ref, ln2b_ref,
                  wt_ref, asm_ref, adm_ref,
                  out_ref, hn_ref, hp_ref, an_ref, bn_ref):
    z = as_ref[...] + ad_ref[...] + ael_ref[...]
    tl = jnp.exp(jnp.maximum(z, 0.2 * z))
    den8 = jnp.dot(r832_ref[...], den_ref[...],
                   preferred_element_type=jnp.float32) + tl
    rmat = rmat_ref[...]
    dfull = jnp.dot(rmat, den8, preferred_element_type=jnp.float32)
    tfull = jnp.dot(rmat, tl, preferred_element_type=jnp.float32)
    g = (acc_ref[...] + tfull * hT_ref[...]) / (dfull + 1e-16) + bias_ref[...]
    t = _ln0(g + mT_ref[...], ln1g_ref[...], ln1b_ref[...])
    f = jnp.maximum(jnp.dot(w1t_ref[...], t, preferred_element_type=jnp.float32)
                    + b1_ref[...], 0.0)
    y = jnp.dot(w2t_ref[...], f, preferred_element_type=jnp.float32) + b2_ref[...]
    out = _ln0(y + t, ln2g_ref[...], ln2b_ref[...])
    out_ref[...] = out
    hn = jnp.dot(wt_ref[...], out, preferred_element_type=jnp.float32)
    hn_ref[...] = hn
    hp_ref[...] = _pack_h(hn)
    an_ref[...] = jnp.dot(asm_ref[...], hn, preferred_element_type=jnp.float32)
    bn_ref[...] = jnp.dot(adm_ref[...], hn, preferred_element_type=jnp.float32)


def _combine(mT, hT, accT, denT, R832, asT, adT, aelT, Rmat, bias,
             w1T, b1, w2T, b2, ln1g, ln1b, ln2g, ln2b, WT, As, Ad):
    nspec = pl.BlockSpec((D, _NB), lambda i: (0, i))
    hspec = pl.BlockSpec((H, _NB), lambda i: (0, i))
    cvec = pl.BlockSpec((D, 1), lambda i: (0, 0))
    return pl.pallas_call(
        _combine_body,
        grid=(NP // _NB,),
        in_specs=[nspec, nspec, nspec,
                  pl.BlockSpec((NW, _NB), lambda i: (0, i)),
                  pl.BlockSpec((H, NW), lambda i: (0, 0)),
                  hspec, hspec, hspec,
                  pl.BlockSpec((D, H), lambda i: (0, 0)),
                  cvec,
                  pl.BlockSpec((4 * D, D), lambda i: (0, 0)),
                  pl.BlockSpec((4 * D, 1), lambda i: (0, 0)),
                  pl.BlockSpec((D, 4 * D), lambda i: (0, 0)),
                  cvec, cvec, cvec, cvec, cvec,
                  pl.BlockSpec((D, D), lambda i: (0, 0)),
                  pl.BlockSpec((H, D), lambda i: (0, 0)),
                  pl.BlockSpec((H, D), lambda i: (0, 0))],
        out_specs=[nspec, nspec, pl.BlockSpec((D // 2, _NB), lambda i: (0, i)),
                   hspec, hspec],
        out_shape=[jax.ShapeDtypeStruct((D, NP), jnp.float32),
                   jax.ShapeDtypeStruct((D, NP), jnp.float32),
                   jax.ShapeDtypeStruct((D // 2, NP), jnp.int32),
                   jax.ShapeDtypeStruct((H, NP), jnp.float32),
                   jax.ShapeDtypeStruct((H, NP), jnp.float32)],
    )(mT, hT, accT, denT, R832, asT, adT, aelT, Rmat, bias, w1T, b1, w2T, b2,
      ln1g, ln1b, ln2g, ln2b, WT, As, Ad)


def kernel(x, edge_index, edge_attr, gat_W, gat_att_src, gat_att_dst,
           gat_lin_edge, gat_att_edge, gat_bias, ffn_w1, ffn_b1, ffn_w2,
           ffn_b2, ln1_g, ln1_b, ln2_g, ln2_b):
    src = edge_index[0]
    dst = edge_index[1]
    sdE = jnp.bitwise_or(src, jnp.left_shift(dst, 16))
    xT = jnp.zeros((D, NP), jnp.float32).at[:, :N].set(x.T)
    eaT = edge_attr.T

    WT = gat_W.T
    le_r = gat_lin_edge.reshape(DE, H, C)
    Pt = jnp.einsum('dhc,hc->hd', le_r, gat_att_edge[0])
    eye8 = jnp.eye(H, dtype=jnp.float32)
    As = (eye8[:, :, None] * gat_att_src[0][:, None, :]).reshape(H, D)
    Ad = (eye8[:, :, None] * gat_att_dst[0][:, None, :]).reshape(H, D)
    Rmat = jnp.repeat(eye8, C, axis=0)
    R832 = jnp.repeat(eye8, NW // H, axis=1)
    bias = gat_bias.reshape(D, 1)

    aeT = _ae_prep(Pt, eaT).reshape(H * E)
    stats = _loopstat_sc(eaT.reshape(DE * E), sdE).reshape(NW + 2, NP)
    aelT = _loopprep(Pt, stats)

    mT = xT
    hT, hP, asT, adT = _head(WT, As, Ad, xT)
    for i in range(L):
        accf, denf = _edge_sc(hP.reshape((D // 2) * NP), asT.reshape(H * NP),
                              adT.reshape(H * NP), aeT, sdE)
        accT = accf.reshape(D, NP)
        denT = denf.reshape(NW, NP)
        mT, hT, hP, asT, adT = _combine(
            mT, hT, accT, denT, R832, asT, adT, aelT, Rmat, bias,
            ffn_w1[i].T, ffn_b1[i].reshape(4 * D, 1),
            ffn_w2[i].T, ffn_b2[i].reshape(D, 1),
            ln1_g[i].reshape(D, 1), ln1_b[i].reshape(D, 1),
            ln2_g[i].reshape(D, 1), ln2_b[i].reshape(D, 1),
            WT, As, Ad)
    return mT[:, :N].T
